# Initial kernel scaffold; baseline (speedup 1.0000x reference)
import jax
import jax.numpy as jnp
from jax.experimental import pallas as pl


def kernel(x, input_embedding, output_embedding):
    raise NotImplementedError("write your pallas kernel here")


# speedup vs baseline: 3.8345x; 3.8345x over previous
import functools

import jax
import jax.numpy as jnp
from jax import lax
from jax.experimental import pallas as pl
from jax.experimental.pallas import tpu as pltpu
from jax.experimental.pallas import tpu_sc as plsc

NC = 2
NS = 16
NW = NC * NS
LANES = 16
DEPTH = 2


def _make_kernel(B, V, D):
    per_w = B // NW
    NG = per_w // LANES
    SUPER = NG // DEPTH
    n_full = D // LANES
    rem = D - n_full * LANES
    tail_base = D - LANES

    mesh = plsc.VectorSubcoreMesh(core_axis_name="c", subcore_axis_name="s")

    @functools.partial(
        pl.kernel,
        mesh=mesh,
        out_type=jax.ShapeDtypeStruct((B,), jnp.float32),
        scratch_types=[
            pltpu.VMEM((per_w,), jnp.int32),
            pltpu.VMEM((per_w,), jnp.int32),
            pltpu.VMEM((DEPTH, LANES, D), jnp.float32),
            pltpu.VMEM((DEPTH, LANES, D), jnp.float32),
            pltpu.VMEM((per_w,), jnp.float32),
            pltpu.SemaphoreType.DMA,
            pltpu.SemaphoreType.DMA,
            pltpu.SemaphoreType.DMA,
            pltpu.SemaphoreType.DMA,
        ],
    )
    def k(x0_hbm, x1_hbm, ine_hbm, oute_hbm, out_hbm,
          idx0_v, idx1_v, rin, rout, res_v, si0, so0, si1, so1):
        wid = lax.axis_index("s") * NC + lax.axis_index("c")
        base = wid * per_w
        lane = lax.iota(jnp.int32, LANES)
        tail_mask = lane >= (LANES - rem)
        perms = [lane ^ kk for kk in (8, 4, 2, 1)]
        sems = [(si0, so0), (si1, so1)]

        pltpu.sync_copy(x0_hbm.at[pl.ds(base, per_w)], idx0_v)
        pltpu.sync_copy(x1_hbm.at[pl.ds(base, per_w)], idx1_v)

        def fire(g, slot, sin, sout):
            iv0 = idx0_v[pl.ds(g * LANES, LANES)]
            iv1 = idx1_v[pl.ds(g * LANES, LANES)]
            for t in range(LANES):
                pltpu.async_copy(
                    ine_hbm.at[pl.ds(iv0[t], 1), :],
                    rin.at[slot, pl.ds(t, 1), :], sin)
                pltpu.async_copy(
                    oute_hbm.at[pl.ds(iv1[t], 1), :],
                    rout.at[slot, pl.ds(t, 1), :], sout)

        def wait_slot(slot, sin, sout):
            pltpu.make_async_copy(
                ine_hbm.at[pl.ds(0, LANES), :], rin.at[slot], sin).wait()
            pltpu.make_async_copy(
                oute_hbm.at[pl.ds(0, LANES), :], rout.at[slot], sout).wait()

        def hsum_all(v):
            for p in perms:
                v = v + jnp.take_along_axis(
                    v, p, axis=0, mode="promise_in_bounds")
            return v

        def compute(g, slot):
            def item(t, resvec):
                acc = (rin[slot, t, pl.ds(0, LANES)]
                       * rout[slot, t, pl.ds(0, LANES)])
                for c in range(1, n_full):
                    acc += (rin[slot, t, pl.ds(c * LANES, LANES)]
                            * rout[slot, t, pl.ds(c * LANES, LANES)])
                tail = (rin[slot, t, pl.ds(tail_base, LANES)]
                        * rout[slot, t, pl.ds(tail_base, LANES)])
                acc += jnp.where(tail_mask, tail, jnp.float32(0.0))
                return jnp.where(lane == t, hsum_all(acc), resvec)

            resvec = lax.fori_loop(
                0, LANES, item, jnp.zeros((LANES,), jnp.float32))
            res_v[pl.ds(g * LANES, LANES)] = resvec

        for s in range(DEPTH):
            fire(s, s, *sems[s])

        def super_body(kk, carry):
            g0 = kk * DEPTH
            for s in range(DEPTH):
                wait_slot(s, *sems[s])
                compute(g0 + s, s)
                fire(g0 + s + DEPTH, s, *sems[s])
            return carry

        lax.fori_loop(0, SUPER - 1, super_body, 0)

        for s in range(DEPTH):
            g = (SUPER - 1) * DEPTH + s
            wait_slot(s, *sems[s])
            compute(g, s)

        pltpu.sync_copy(res_v, out_hbm.at[pl.ds(base, per_w)])

    return k


@jax.jit
def kernel(x, input_embedding, output_embedding):
    B = x.shape[0]
    V, D = input_embedding.shape
    x0 = x[:, 0]
    x1 = x[:, 1]
    k = _make_kernel(B, V, D)
    return k(x0, x1, input_embedding, output_embedding)
